# Initial kernel scaffold; baseline (speedup 1.0000x reference)
import jax
import jax.numpy as jnp
from jax.experimental import pallas as pl


def kernel(x, edge_index, edge_attr, Wx, bx, We, be):
    raise NotImplementedError("write your pallas kernel here")


# speedup vs baseline: 1.2364x; 1.2364x over previous
import functools

import jax
import jax.numpy as jnp
from jax import lax
from jax.experimental import pallas as pl
from jax.experimental.pallas import tpu as pltpu
from jax.experimental.pallas import tpu_sc as plsc

_NC = 2
_NS = 16
_NW = _NC * _NS

_CHUNK = 80


def _mm_bias_body(x_ref, w_ref, b_ref, o_ref):
    o_ref[...] = (
        jnp.dot(x_ref[...], w_ref[...], preferred_element_type=jnp.float32)
        + b_ref[...]
    )


def _tc_matmul_bias(x, wt, b, blk):
    n, d = x.shape
    dout = wt.shape[1]
    return pl.pallas_call(
        _mm_bias_body,
        grid=(n // blk,),
        in_specs=[
            pl.BlockSpec((blk, d), lambda i: (i, 0)),
            pl.BlockSpec((d, dout), lambda i: (0, 0)),
            pl.BlockSpec((1, dout), lambda i: (0, 0)),
        ],
        out_specs=pl.BlockSpec((blk, dout), lambda i: (i, 0)),
        out_shape=jax.ShapeDtypeStruct((n, dout), jnp.float32),
    )(x, wt, b)


def _make_sc_gather(n_edges, d_out_e, d_out_x):
    d_out = d_out_e + d_out_x
    per_w = n_edges // _NW
    n_chunks = per_w // _CHUNK
    assert per_w % _CHUNK == 0 and _CHUNK % 8 == 0

    mesh = plsc.VectorSubcoreMesh(core_axis_name="c", subcore_axis_name="s")

    @functools.partial(
        pl.kernel,
        mesh=mesh,
        out_type=jax.ShapeDtypeStruct((n_edges, d_out), jnp.float32),
        scratch_types=[
            pltpu.VMEM((_CHUNK,), jnp.int32),
            pltpu.VMEM((_CHUNK,), jnp.int32),
            pltpu.VMEM((_CHUNK, d_out_x), jnp.float32),
            pltpu.VMEM((_CHUNK, d_out_x), jnp.float32),
            pltpu.VMEM((_CHUNK, d_out_e), jnp.float32),
            pltpu.VMEM((_CHUNK, d_out), jnp.float32),
            pltpu.SemaphoreType.DMA,
            pltpu.SemaphoreType.DMA,
        ],
    )
    def sc_gather(xw_hbm, src_hbm, dst_hbm, ew_hbm, out_hbm,
                  idx_s, idx_d, buf_s, buf_d, buf_e, pack, sem_s, sem_d):
        wid = lax.axis_index("s") * _NC + lax.axis_index("c")
        wbase = wid * per_w

        def chunk_body(j, carry):
            base = pl.multiple_of(wbase + j * _CHUNK, 8)
            pltpu.sync_copy(src_hbm.at[pl.ds(base, _CHUNK)], idx_s)
            pltpu.sync_copy(dst_hbm.at[pl.ds(base, _CHUNK)], idx_d)
            cp_s = pltpu.async_copy(xw_hbm.at[idx_s], buf_s, sem_s)
            cp_d = pltpu.async_copy(xw_hbm.at[idx_d], buf_d, sem_d)
            pltpu.sync_copy(ew_hbm.at[pl.ds(base, _CHUNK)], buf_e)
            cp_s.wait()
            cp_d.wait()

            def row_body(i, c):
                pack[i, pl.ds(0, d_out_e)] = buf_e[i, :]
                for k in range(d_out_x // 16):
                    pack[i, pl.ds(d_out_e + 16 * k, 16)] = (
                        buf_s[i, pl.ds(16 * k, 16)] + buf_d[i, pl.ds(16 * k, 16)]
                    )
                return c

            lax.fori_loop(0, _CHUNK, row_body, 0)
            pltpu.sync_copy(pack, out_hbm.at[pl.ds(base, _CHUNK)])
            return carry

        lax.fori_loop(0, n_chunks, chunk_body, 0)

    return sc_gather


def kernel(x, edge_index, edge_attr, Wx, bx, We, be):
    n_edges, d_edge = edge_attr.shape
    d_out_x = Wx.shape[0]
    d_out_e = We.shape[0]

    src = edge_index[0].astype(jnp.int32)
    dst = edge_index[1].astype(jnp.int32)

    xw = _tc_matmul_bias(x, Wx.T, (0.5 * bx)[None, :], blk=2000)

    packf = 128 // d_edge
    we_bd = jnp.kron(jnp.eye(packf, dtype=We.dtype), We.T)
    ew = _tc_matmul_bias(
        edge_attr.reshape(n_edges // packf, packf * d_edge),
        we_bd,
        jnp.tile(be, packf)[None, :],
        blk=4000,
    ).reshape(n_edges, d_out_e)

    sc = _make_sc_gather(n_edges, d_out_e, d_out_x)
    return sc(xw, src, dst, ew)


# speedup vs baseline: 1.8038x; 1.4589x over previous
import functools

import jax
import jax.numpy as jnp
from jax import lax
from jax.experimental import pallas as pl
from jax.experimental.pallas import tpu as pltpu
from jax.experimental.pallas import tpu_sc as plsc

_NC = 2
_NS = 16
_NW = _NC * _NS

_CHUNK = 40


def _mm_bias_body(x_ref, w_ref, b_ref, o_ref):
    o_ref[...] = (
        jnp.dot(x_ref[...], w_ref[...], preferred_element_type=jnp.float32)
        + b_ref[...]
    )


def _tc_matmul_bias(x, wt, b, blk):
    n, d = x.shape
    dout = wt.shape[1]
    return pl.pallas_call(
        _mm_bias_body,
        grid=(n // blk,),
        in_specs=[
            pl.BlockSpec((blk, d), lambda i: (i, 0)),
            pl.BlockSpec((d, dout), lambda i: (0, 0)),
            pl.BlockSpec((1, dout), lambda i: (0, 0)),
        ],
        out_specs=pl.BlockSpec((blk, dout), lambda i: (i, 0)),
        out_shape=jax.ShapeDtypeStruct((n, dout), jnp.float32),
    )(x, wt, b)


def _make_sc_gather(n_edges, d_out_e, d_out_x):
    d_out = d_out_e + d_out_x
    per_w = n_edges // _NW
    n_chunks = per_w // _CHUNK
    assert per_w % _CHUNK == 0 and _CHUNK % 8 == 0 and n_chunks % 2 == 0

    mesh = plsc.VectorSubcoreMesh(core_axis_name="c", subcore_axis_name="s")

    @functools.partial(
        pl.kernel,
        mesh=mesh,
        out_type=jax.ShapeDtypeStruct((n_edges, d_out), jnp.float32),
        scratch_types=[
            pltpu.VMEM((n_chunks, _CHUNK), jnp.int32),
            pltpu.VMEM((n_chunks, _CHUNK), jnp.int32),
            pltpu.VMEM((_CHUNK, d_out_x), jnp.float32),
            pltpu.VMEM((_CHUNK, d_out_x), jnp.float32),
            pltpu.VMEM((_CHUNK, d_out_x), jnp.float32),
            pltpu.VMEM((_CHUNK, d_out_x), jnp.float32),
            pltpu.VMEM((_CHUNK, d_out_e), jnp.float32),
            pltpu.VMEM((_CHUNK, d_out_e), jnp.float32),
            pltpu.VMEM((_CHUNK, d_out), jnp.float32),
            pltpu.VMEM((_CHUNK, d_out), jnp.float32),
            pltpu.SemaphoreType.DMA,
            pltpu.SemaphoreType.DMA,
            pltpu.SemaphoreType.DMA,
            pltpu.SemaphoreType.DMA,
            pltpu.SemaphoreType.DMA,
            pltpu.SemaphoreType.DMA,
            pltpu.SemaphoreType.DMA,
            pltpu.SemaphoreType.DMA,
        ],
    )
    def sc_gather(xw_hbm, src_hbm, dst_hbm, ew_hbm, out_hbm,
                  idx_s, idx_d,
                  buf_s0, buf_s1, buf_d0, buf_d1, buf_e0, buf_e1,
                  pack0, pack1,
                  sem_s0, sem_s1, sem_d0, sem_d1,
                  sem_e0, sem_e1, sem_o0, sem_o1):
        wid = lax.axis_index("s") * _NC + lax.axis_index("c")
        wbase = wid * per_w
        buf_s = (buf_s0, buf_s1)
        buf_d = (buf_d0, buf_d1)
        buf_e = (buf_e0, buf_e1)
        pack = (pack0, pack1)
        sem_s = (sem_s0, sem_s1)
        sem_d = (sem_d0, sem_d1)
        sem_e = (sem_e0, sem_e1)
        sem_o = (sem_o0, sem_o1)

        pltpu.sync_copy(src_hbm.at[wid], idx_s)
        pltpu.sync_copy(dst_hbm.at[wid], idx_d)

        def issue(cj, b):
            base = pl.multiple_of(wbase + cj * _CHUNK, 8)
            pltpu.async_copy(xw_hbm.at[idx_s.at[cj]], buf_s[b], sem_s[b])
            pltpu.async_copy(xw_hbm.at[idx_d.at[cj]], buf_d[b], sem_d[b])
            pltpu.async_copy(ew_hbm.at[pl.ds(base, _CHUNK)], buf_e[b], sem_e[b])

        for b in range(2):
            issue(b, b)

        def chunk_body(j, carry):
            for b in range(2):
                cj = 2 * j + b
                base = pl.multiple_of(wbase + cj * _CHUNK, 8)
                pltpu.make_async_copy(
                    xw_hbm.at[idx_s.at[0]], buf_s[b], sem_s[b]).wait()
                pltpu.make_async_copy(
                    xw_hbm.at[idx_d.at[0]], buf_d[b], sem_d[b]).wait()
                pltpu.make_async_copy(
                    ew_hbm.at[pl.ds(0, _CHUNK)], buf_e[b], sem_e[b]).wait()

                @pl.when(j >= 1)
                def _():
                    pltpu.make_async_copy(
                        pack[b], out_hbm.at[pl.ds(0, _CHUNK)], sem_o[b]).wait()

                def row_body(i, c):
                    pack[b][i, pl.ds(0, d_out_e)] = buf_e[b][i, :]
                    for k in range(d_out_x // 16):
                        pack[b][i, pl.ds(d_out_e + 16 * k, 16)] = (
                            buf_s[b][i, pl.ds(16 * k, 16)]
                            + buf_d[b][i, pl.ds(16 * k, 16)]
                        )
                    return c

                lax.fori_loop(0, _CHUNK, row_body, 0, unroll=2)

                pltpu.async_copy(
                    pack[b], out_hbm.at[pl.ds(base, _CHUNK)], sem_o[b])

                @pl.when(j < (n_chunks // 2 - 1))
                def _():
                    issue(cj + 2, b)
            return carry

        lax.fori_loop(0, n_chunks // 2, chunk_body, 0)

        for b in range(2):
            pltpu.make_async_copy(
                pack[b], out_hbm.at[pl.ds(0, _CHUNK)], sem_o[b]).wait()

    return sc_gather


def kernel(x, edge_index, edge_attr, Wx, bx, We, be):
    n_edges, d_edge = edge_attr.shape
    d_out_x = Wx.shape[0]
    d_out_e = We.shape[0]
    per_w = n_edges // _NW
    n_chunks = per_w // _CHUNK

    src = edge_index[0].astype(jnp.int32).reshape(_NW, n_chunks, _CHUNK)
    dst = edge_index[1].astype(jnp.int32).reshape(_NW, n_chunks, _CHUNK)

    xw = _tc_matmul_bias(x, Wx.T, (0.5 * bx)[None, :], blk=2000)

    packf = 128 // d_edge
    we_bd = jnp.kron(jnp.eye(packf, dtype=We.dtype), We.T)
    ew = _tc_matmul_bias(
        edge_attr.reshape(n_edges // packf, packf * d_edge),
        we_bd,
        jnp.tile(be, packf)[None, :],
        blk=4000,
    ).reshape(n_edges, d_out_e)

    sc = _make_sc_gather(n_edges, d_out_e, d_out_x)
    return sc(xw, src, dst, ew)


# speedup vs baseline: 2.0362x; 1.1288x over previous
import functools

import jax
import jax.numpy as jnp
from jax import lax
from jax.experimental import pallas as pl
from jax.experimental.pallas import tpu as pltpu
from jax.experimental.pallas import tpu_sc as plsc

_NC = 2
_NS = 16
_NW = _NC * _NS

_CHUNK = 40


def _mm_bias_body(x_ref, w_ref, b_ref, o_ref):
    o_ref[...] = (
        jnp.dot(x_ref[...], w_ref[...], preferred_element_type=jnp.float32)
        + b_ref[...]
    )


def _tc_matmul_bias(x, wt, b, blk):
    n, d = x.shape
    dout = wt.shape[1]
    return pl.pallas_call(
        _mm_bias_body,
        grid=(n // blk,),
        in_specs=[
            pl.BlockSpec((blk, d), lambda i: (i, 0)),
            pl.BlockSpec((d, dout), lambda i: (0, 0)),
            pl.BlockSpec((1, dout), lambda i: (0, 0)),
        ],
        out_specs=pl.BlockSpec((blk, dout), lambda i: (i, 0)),
        out_shape=jax.ShapeDtypeStruct((n, dout), jnp.float32),
    )(x, wt, b)


def _make_sc_gather(n_edges, d_out_e, d_out_x):
    d_out = d_out_e + d_out_x
    per_w = n_edges // _NW
    n_chunks = per_w // _CHUNK
    assert per_w % _CHUNK == 0 and _CHUNK % 8 == 0 and n_chunks % 2 == 0

    mesh = plsc.VectorSubcoreMesh(core_axis_name="c", subcore_axis_name="s")

    @functools.partial(
        pl.kernel,
        mesh=mesh,
        out_type=jax.ShapeDtypeStruct((n_edges, d_out), jnp.float32),
        scratch_types=[
            pltpu.VMEM((per_w,), jnp.int32),
            pltpu.VMEM((per_w,), jnp.int32),
            pltpu.VMEM((_CHUNK, d_out_x), jnp.float32),
            pltpu.VMEM((_CHUNK, d_out_x), jnp.float32),
            pltpu.VMEM((_CHUNK, d_out_x), jnp.float32),
            pltpu.VMEM((_CHUNK, d_out_x), jnp.float32),
            pltpu.VMEM((_CHUNK * d_out_e,), jnp.float32),
            pltpu.VMEM((_CHUNK * d_out_e,), jnp.float32),
            pltpu.VMEM((_CHUNK, d_out), jnp.float32),
            pltpu.VMEM((_CHUNK, d_out), jnp.float32),
            pltpu.SemaphoreType.DMA,
            pltpu.SemaphoreType.DMA,
            pltpu.SemaphoreType.DMA,
            pltpu.SemaphoreType.DMA,
            pltpu.SemaphoreType.DMA,
            pltpu.SemaphoreType.DMA,
            pltpu.SemaphoreType.DMA,
            pltpu.SemaphoreType.DMA,
        ],
    )
    def sc_gather(xw_hbm, src_hbm, dst_hbm, ew_hbm, out_hbm,
                  idx_s, idx_d,
                  buf_s0, buf_s1, buf_d0, buf_d1, buf_e0, buf_e1,
                  pack0, pack1,
                  sem_s0, sem_s1, sem_d0, sem_d1,
                  sem_e0, sem_e1, sem_o0, sem_o1):
        wid = lax.axis_index("s") * _NC + lax.axis_index("c")
        wbase = wid * per_w
        buf_s = (buf_s0, buf_s1)
        buf_d = (buf_d0, buf_d1)
        buf_e = (buf_e0, buf_e1)
        pack = (pack0, pack1)
        sem_s = (sem_s0, sem_s1)
        sem_d = (sem_d0, sem_d1)
        sem_e = (sem_e0, sem_e1)
        sem_o = (sem_o0, sem_o1)

        pltpu.sync_copy(src_hbm.at[pl.ds(wbase, per_w)], idx_s)
        pltpu.sync_copy(dst_hbm.at[pl.ds(wbase, per_w)], idx_d)

        def issue(cj, b):
            off = pl.multiple_of(cj * _CHUNK, 8)
            ebase = pl.multiple_of((wbase + cj * _CHUNK) * d_out_e, 8)
            pltpu.async_copy(
                xw_hbm.at[idx_s.at[pl.ds(off, _CHUNK)]], buf_s[b], sem_s[b])
            pltpu.async_copy(
                xw_hbm.at[idx_d.at[pl.ds(off, _CHUNK)]], buf_d[b], sem_d[b])
            pltpu.async_copy(
                ew_hbm.at[pl.ds(ebase, _CHUNK * d_out_e)], buf_e[b], sem_e[b])

        for b in range(2):
            issue(b, b)

        def chunk_body(j, carry):
            for b in range(2):
                cj = 2 * j + b
                base = pl.multiple_of(wbase + cj * _CHUNK, 8)
                pltpu.make_async_copy(
                    xw_hbm.at[idx_s.at[pl.ds(0, _CHUNK)]], buf_s[b],
                    sem_s[b]).wait()
                pltpu.make_async_copy(
                    xw_hbm.at[idx_d.at[pl.ds(0, _CHUNK)]], buf_d[b],
                    sem_d[b]).wait()
                pltpu.make_async_copy(
                    ew_hbm.at[pl.ds(0, _CHUNK * d_out_e)], buf_e[b],
                    sem_e[b]).wait()

                @pl.when(j >= 1)
                def _():
                    pltpu.make_async_copy(
                        pack[b], out_hbm.at[pl.ds(0, _CHUNK)], sem_o[b]).wait()

                def row_body(i, c):
                    pack[b][i, pl.ds(0, d_out_e)] = (
                        buf_e[b][pl.ds(i * d_out_e, d_out_e)]
                    )
                    for k in range(d_out_x // 16):
                        pack[b][i, pl.ds(d_out_e + 16 * k, 16)] = (
                            buf_s[b][i, pl.ds(16 * k, 16)]
                            + buf_d[b][i, pl.ds(16 * k, 16)]
                        )
                    return c

                lax.fori_loop(0, _CHUNK, row_body, 0, unroll=2)

                pltpu.async_copy(
                    pack[b], out_hbm.at[pl.ds(base, _CHUNK)], sem_o[b])

                @pl.when(j < (n_chunks // 2 - 1))
                def _():
                    issue(cj + 2, b)
            return carry

        lax.fori_loop(0, n_chunks // 2, chunk_body, 0)

        for b in range(2):
            pltpu.make_async_copy(
                pack[b], out_hbm.at[pl.ds(0, _CHUNK)], sem_o[b]).wait()

    return sc_gather


def kernel(x, edge_index, edge_attr, Wx, bx, We, be):
    n_edges, d_edge = edge_attr.shape
    d_out_x = Wx.shape[0]
    d_out_e = We.shape[0]
    src = edge_index[0].astype(jnp.int32)
    dst = edge_index[1].astype(jnp.int32)

    xw = _tc_matmul_bias(x, Wx.T, (0.5 * bx)[None, :], blk=2000)

    packf = 128 // d_edge
    we_bd = jnp.kron(jnp.eye(packf, dtype=We.dtype), We.T)
    ew = _tc_matmul_bias(
        edge_attr.reshape(n_edges // packf, packf * d_edge),
        we_bd,
        jnp.tile(be, packf)[None, :],
        blk=4000,
    ).reshape(-1)

    sc = _make_sc_gather(n_edges, d_out_e, d_out_x)
    return sc(xw, src, dst, ew)


# speedup vs baseline: 2.6435x; 1.2983x over previous
import functools

import jax
import jax.numpy as jnp
from jax import lax
from jax.experimental import pallas as pl
from jax.experimental.pallas import tpu as pltpu
from jax.experimental.pallas import tpu_sc as plsc

_NC = 2
_NS = 16
_NW = _NC * _NS

_CHUNK = 40


def _mm_bias_body(x_ref, w_ref, b_ref, o_ref):
    o_ref[...] = (
        jnp.dot(x_ref[...], w_ref[...], preferred_element_type=jnp.float32)
        + b_ref[...]
    )


def _tc_matmul_bias(x, wt, b, blk):
    n, d = x.shape
    dout = wt.shape[1]
    return pl.pallas_call(
        _mm_bias_body,
        grid=(n // blk,),
        in_specs=[
            pl.BlockSpec((blk, d), lambda i: (i, 0)),
            pl.BlockSpec((d, dout), lambda i: (0, 0)),
            pl.BlockSpec((1, dout), lambda i: (0, 0)),
        ],
        out_specs=pl.BlockSpec((blk, dout), lambda i: (i, 0)),
        out_shape=jax.ShapeDtypeStruct((n, dout), jnp.float32),
    )(x, wt, b)


def _make_sc_gather(n_edges, d_out_e, d_out_x):
    d_out = d_out_e + d_out_x
    per_w = n_edges // _NW
    n_chunks = per_w // _CHUNK
    assert per_w % _CHUNK == 0 and _CHUNK % 8 == 0 and n_chunks % 2 == 0

    mesh = plsc.VectorSubcoreMesh(core_axis_name="c", subcore_axis_name="s")

    @functools.partial(
        pl.kernel,
        mesh=mesh,
        out_type=jax.ShapeDtypeStruct((n_edges, d_out), jnp.float32),
        scratch_types=[
            pltpu.VMEM((per_w,), jnp.int32),
            pltpu.VMEM((per_w,), jnp.int32),
            pltpu.VMEM((_CHUNK, d_out_x), jnp.float32),
            pltpu.VMEM((_CHUNK, d_out_x), jnp.float32),
            pltpu.VMEM((_CHUNK, d_out_x), jnp.float32),
            pltpu.VMEM((_CHUNK, d_out_x), jnp.float32),
            pltpu.VMEM((_CHUNK * d_out_e,), jnp.float32),
            pltpu.VMEM((_CHUNK * d_out_e,), jnp.float32),
            pltpu.VMEM((_CHUNK, d_out), jnp.float32),
            pltpu.VMEM((_CHUNK, d_out), jnp.float32),
            pltpu.SemaphoreType.DMA,
            pltpu.SemaphoreType.DMA,
            pltpu.SemaphoreType.DMA,
            pltpu.SemaphoreType.DMA,
            pltpu.SemaphoreType.DMA,
            pltpu.SemaphoreType.DMA,
            pltpu.SemaphoreType.DMA,
            pltpu.SemaphoreType.DMA,
        ],
    )
    def sc_gather(xw_hbm, src_hbm, dst_hbm, ew_hbm, out_hbm,
                  idx_s, idx_d,
                  buf_s0, buf_s1, buf_d0, buf_d1, buf_e0, buf_e1,
                  pack0, pack1,
                  sem_s0, sem_s1, sem_d0, sem_d1,
                  sem_e0, sem_e1, sem_o0, sem_o1):
        wid = lax.axis_index("s") * _NC + lax.axis_index("c")
        wbase = wid * per_w
        buf_s = (buf_s0, buf_s1)
        buf_d = (buf_d0, buf_d1)
        buf_e = (buf_e0, buf_e1)
        pack = (pack0, pack1)
        sem_s = (sem_s0, sem_s1)
        sem_d = (sem_d0, sem_d1)
        sem_e = (sem_e0, sem_e1)
        sem_o = (sem_o0, sem_o1)

        pltpu.sync_copy(src_hbm.at[pl.ds(wbase, per_w)], idx_s)
        pltpu.sync_copy(dst_hbm.at[pl.ds(wbase, per_w)], idx_d)

        def issue(cj, b):
            off = pl.multiple_of(cj * _CHUNK, 8)
            ebase = pl.multiple_of((wbase + cj * _CHUNK) * d_out_e, 8)
            pltpu.async_copy(
                xw_hbm.at[idx_s.at[pl.ds(off, _CHUNK)]], buf_s[b], sem_s[b])
            pltpu.async_copy(
                xw_hbm.at[idx_d.at[pl.ds(off, _CHUNK)]], buf_d[b], sem_d[b])
            pltpu.async_copy(
                ew_hbm.at[pl.ds(ebase, _CHUNK * d_out_e)], buf_e[b], sem_e[b])

        for b in range(2):
            issue(b, b)

        def chunk_body(j, carry):
            for b in range(2):
                cj = 2 * j + b
                base = pl.multiple_of(wbase + cj * _CHUNK, 8)
                pltpu.make_async_copy(
                    xw_hbm.at[idx_s.at[pl.ds(0, _CHUNK)]], buf_s[b],
                    sem_s[b]).wait()
                pltpu.make_async_copy(
                    xw_hbm.at[idx_d.at[pl.ds(0, _CHUNK)]], buf_d[b],
                    sem_d[b]).wait()
                pltpu.make_async_copy(
                    ew_hbm.at[pl.ds(0, _CHUNK * d_out_e)], buf_e[b],
                    sem_e[b]).wait()

                @pl.when(j >= 1)
                def _():
                    pltpu.make_async_copy(
                        pack[b], out_hbm.at[pl.ds(0, _CHUNK)], sem_o[b]).wait()

                @plsc.parallel_loop(0, _CHUNK, unroll=2)
                def _(i):
                    pack[b][i, pl.ds(0, d_out_e)] = (
                        buf_e[b][pl.ds(i * d_out_e, d_out_e)]
                    )
                    for k in range(d_out_x // 16):
                        pack[b][i, pl.ds(d_out_e + 16 * k, 16)] = (
                            buf_s[b][i, pl.ds(16 * k, 16)]
                            + buf_d[b][i, pl.ds(16 * k, 16)]
                        )

                pltpu.async_copy(
                    pack[b], out_hbm.at[pl.ds(base, _CHUNK)], sem_o[b])

                @pl.when(j < (n_chunks // 2 - 1))
                def _():
                    issue(cj + 2, b)
            return carry

        lax.fori_loop(0, n_chunks // 2, chunk_body, 0)

        for b in range(2):
            pltpu.make_async_copy(
                pack[b], out_hbm.at[pl.ds(0, _CHUNK)], sem_o[b]).wait()

    return sc_gather


def kernel(x, edge_index, edge_attr, Wx, bx, We, be):
    n_edges, d_edge = edge_attr.shape
    d_out_x = Wx.shape[0]
    d_out_e = We.shape[0]
    src = edge_index[0].astype(jnp.int32)
    dst = edge_index[1].astype(jnp.int32)

    xw = _tc_matmul_bias(x, Wx.T, (0.5 * bx)[None, :], blk=2000)

    packf = 128 // d_edge
    we_bd = jnp.kron(jnp.eye(packf, dtype=We.dtype), We.T)
    ew = _tc_matmul_bias(
        edge_attr.reshape(n_edges // packf, packf * d_edge),
        we_bd,
        jnp.tile(be, packf)[None, :],
        blk=4000,
    ).reshape(-1)

    sc = _make_sc_gather(n_edges, d_out_e, d_out_x)
    return sc(xw, src, dst, ew)
